# Initial kernel scaffold; baseline (speedup 1.0000x reference)
import jax
import jax.numpy as jnp
from jax.experimental import pallas as pl


def kernel(past_test, past_question, past_tag, past_testTag, num0, num1, num2, T_test, T_q, T_tag, T_tt, W_cat, b_cat, g_cat, beta_cat, W_num, b_num, g_num, beta_num, g_out, beta_out):
    raise NotImplementedError("write your pallas kernel here")


# speedup vs baseline: 2.9654x; 2.9654x over previous
import functools

import jax
import jax.numpy as jnp
from jax import lax
from jax.experimental import pallas as pl
from jax.experimental.pallas import tpu as pltpu
from jax.experimental.pallas import tpu_sc as plsc

NC, NS, LANES = 2, 16, 16
NW = NC * NS
B0 = 128
EPS = 1e-6
HID = 64
HALF = HID // 2
INTD = 21


def _proj_body(tt, tq, tg, t4, w, pt, pq, pg, p4):
    pt[...] = jnp.dot(tt[...], w[0], preferred_element_type=jnp.float32)
    pq[...] = jnp.dot(tq[...], w[1], preferred_element_type=jnp.float32)
    pg[...] = jnp.dot(tg[...], w[2], preferred_element_type=jnp.float32)
    p4[...] = jnp.dot(t4[...], w[3], preferred_element_type=jnp.float32)


def _project_tables(T_test, T_q, T_tag, T_tt, W_cat):
    w = W_cat.reshape(4, INTD, HALF)
    outs = [
        jax.ShapeDtypeStruct((t.shape[0], HALF), jnp.float32)
        for t in (T_test, T_q, T_tag, T_tt)
    ]
    return pl.pallas_call(_proj_body, out_shape=outs)(T_test, T_q, T_tag, T_tt, w)


def _rsqrt(x):
    i = lax.bitcast_convert_type(x, jnp.int32)
    i = jnp.int32(0x5F3759DF) - lax.shift_right_arithmetic(i, 1)
    y = lax.bitcast_convert_type(i, jnp.float32)
    for _ in range(3):
        y = y * (1.5 - 0.5 * x * y * y)
    return y


_GDN = lax.GatherDimensionNumbers(
    offset_dims=(), collapsed_slice_dims=(0,), start_index_map=(0,))


def _perm(v, idx):
    return lax.gather(v, idx[:, None], _GDN, slice_sizes=(1,),
                      mode=lax.GatherScatterMode.PROMISE_IN_BOUNDS)


def _bsum(v):
    lanes = lax.iota(jnp.int32, LANES)
    for k in (8, 4, 2, 1):
        v = v + _perm(v, lanes ^ k)
    return v


def _ln(vs, gs, bs, n):
    s1 = vs[0]
    s2 = vs[0] * vs[0]
    for v in vs[1:]:
        s1 = s1 + v
        s2 = s2 + v * v
    s1 = _bsum(s1)[0]
    s2 = _bsum(s2)[0]
    m = s1 * (1.0 / n)
    var = s2 * (1.0 / n) - m * m
    rs = _rsqrt(var + EPS)
    c = -m * rs
    return [(v * rs + c) * g + b for v, g, b in zip(vs, gs, bs)]


def _sc_body(nblk, it, iq, ig, i4, numf, pt, pq, pg, p4, pars, out,
             idx0, idx1, idx2, idx3, rows0, rows1, rows2, rows3,
             numv, parv, outv, sem):
    wid = lax.axis_index("c") * NS + lax.axis_index("s")
    tw = nblk * B0

    pltpu.sync_copy(pars, parv)
    ld = lambda o: parv[pl.ds(o, LANES)]
    w00, w01 = ld(0), ld(16)
    w10, w11 = ld(32), ld(48)
    w20, w21 = ld(64), ld(80)
    gc = (ld(96), ld(112))
    bc = (ld(128), ld(144))
    gn = (ld(160), ld(176))
    bn = (ld(192), ld(208))
    go = (ld(224), ld(240), ld(256), ld(272))
    bo = (ld(288), ld(304), ld(320), ld(336))

    def block(g, _):
        base = wid * tw + g * B0
        pltpu.sync_copy(it.at[pl.ds(base, B0)], idx0)
        pltpu.sync_copy(iq.at[pl.ds(base, B0)], idx1)
        pltpu.sync_copy(ig.at[pl.ds(base, B0)], idx2)
        pltpu.sync_copy(i4.at[pl.ds(base, B0)], idx3)
        c0 = pltpu.async_copy(pt.at[idx0], rows0, sem)
        c1 = pltpu.async_copy(pq.at[idx1], rows1, sem)
        c2 = pltpu.async_copy(pg.at[idx2], rows2, sem)
        c3 = pltpu.async_copy(p4.at[idx3], rows3, sem)
        pltpu.sync_copy(numf.at[pl.ds(3 * base, 3 * B0)], numv.at[pl.ds(0, 3 * B0)])
        c0.wait()
        c1.wait()
        c2.wait()
        c3.wait()

        def token(t, _):
            a0 = (rows0[t, pl.ds(0, LANES)] + rows1[t, pl.ds(0, LANES)]
                  + rows2[t, pl.ds(0, LANES)] + rows3[t, pl.ds(0, LANES)])
            a1 = (rows0[t, pl.ds(LANES, LANES)] + rows1[t, pl.ds(LANES, LANES)]
                  + rows2[t, pl.ds(LANES, LANES)] + rows3[t, pl.ds(LANES, LANES)])
            cat0, cat1 = _ln([a0, a1], gc, bc, HALF)
            nv = numv[pl.ds(3 * t, LANES)]
            n0, n1, n2 = nv[0], nv[1], nv[2]
            u0 = n0 * w00 + n1 * w10 + n2 * w20
            u1 = n0 * w01 + n1 * w11 + n2 * w21
            nm0, nm1 = _ln([u0, u1], gn, bn, HALF)
            o = _ln([cat0, cat1, nm0, nm1], go, bo, HID)
            outv[t, pl.ds(0, LANES)] = o[0]
            outv[t, pl.ds(LANES, LANES)] = o[1]
            outv[t, pl.ds(2 * LANES, LANES)] = o[2]
            outv[t, pl.ds(3 * LANES, LANES)] = o[3]
            return 0

        lax.fori_loop(0, B0, token, 0)
        pltpu.sync_copy(outv, out.at[pl.ds(base, B0), :])
        return 0

    lax.fori_loop(0, nblk, block, 0)


def kernel(past_test, past_question, past_tag, past_testTag, num0, num1, num2,
           T_test, T_q, T_tag, T_tt, W_cat, b_cat, g_cat, beta_cat,
           W_num, b_num, g_num, beta_num, g_out, beta_out):
    B, L = past_test.shape
    T = B * L
    assert T % (NW * B0) == 0
    nblk = T // (NW * B0)

    pt, pq, pg, p4 = _project_tables(T_test, T_q, T_tag, T_tt, W_cat)

    it = past_test.reshape(T)
    iq = past_question.reshape(T)
    ig = past_tag.reshape(T)
    i4 = past_testTag.reshape(T)
    numf = jnp.concatenate([
        num0.astype(jnp.float32).reshape(T),
        num1.astype(jnp.float32).reshape(T),
        num2.astype(jnp.float32).reshape(T),
    ])
    pars = jnp.concatenate([
        W_num.astype(jnp.float32).reshape(3 * HALF),
        g_cat, beta_cat, g_num, beta_num, g_out, beta_out,
    ])

    mesh = plsc.VectorSubcoreMesh(core_axis_name="c", subcore_axis_name="s",
                                  num_cores=NC, num_subcores=NS)
    run = pl.kernel(
        functools.partial(_sc_body, nblk),
        out_type=jax.ShapeDtypeStruct((T, HID), jnp.float32),
        mesh=mesh,
        compiler_params=pltpu.CompilerParams(use_tc_tiling_on_sc=False),
        scratch_types=[
            pltpu.VMEM((B0,), jnp.int32),
            pltpu.VMEM((B0,), jnp.int32),
            pltpu.VMEM((B0,), jnp.int32),
            pltpu.VMEM((B0,), jnp.int32),
            pltpu.VMEM((B0, HALF), jnp.float32),
            pltpu.VMEM((B0, HALF), jnp.float32),
            pltpu.VMEM((B0, HALF), jnp.float32),
            pltpu.VMEM((B0, HALF), jnp.float32),
            pltpu.VMEM((3 * B0 + LANES,), jnp.float32),
            pltpu.VMEM((11 * 32,), jnp.float32),
            pltpu.VMEM((B0, HID), jnp.float32),
            pltpu.SemaphoreType.DMA,
        ],
    )
    out = run(it, iq, ig, i4, numf, pt, pq, pg, p4, pars)
    return out.reshape(B, L, HID)


# speedup vs baseline: 2.9662x; 1.0003x over previous
import functools

import jax
import jax.numpy as jnp
from jax import lax
from jax.experimental import pallas as pl
from jax.experimental.pallas import tpu as pltpu
from jax.experimental.pallas import tpu_sc as plsc

NC, NS, LANES = 2, 16, 16
NW = NC * NS
B0 = 128
EPS = 1e-6
HID = 64
HALF = HID // 2
INTD = 21


def _proj_body(tt, tq, tg, t4, w, pt, pq, pg, p4):
    pt[...] = jnp.dot(tt[...], w[0], preferred_element_type=jnp.float32)
    pq[...] = jnp.dot(tq[...], w[1], preferred_element_type=jnp.float32)
    pg[...] = jnp.dot(tg[...], w[2], preferred_element_type=jnp.float32)
    p4[...] = jnp.dot(t4[...], w[3], preferred_element_type=jnp.float32)


def _project_tables(T_test, T_q, T_tag, T_tt, W_cat):
    w = W_cat.reshape(4, INTD, HALF)
    outs = [
        jax.ShapeDtypeStruct((t.shape[0], HALF), jnp.float32)
        for t in (T_test, T_q, T_tag, T_tt)
    ]
    return pl.pallas_call(_proj_body, out_shape=outs)(T_test, T_q, T_tag, T_tt, w)


def _rsqrt(x):
    i = lax.bitcast_convert_type(x, jnp.int32)
    i = jnp.int32(0x5F3759DF) - lax.shift_right_arithmetic(i, 1)
    y = lax.bitcast_convert_type(i, jnp.float32)
    for _ in range(3):
        y = y * (1.5 - 0.5 * x * y * y)
    return y


_GDN = lax.GatherDimensionNumbers(
    offset_dims=(), collapsed_slice_dims=(0,), start_index_map=(0,))


def _perm(v, idx):
    return lax.gather(v, idx[:, None], _GDN, slice_sizes=(1,),
                      mode=lax.GatherScatterMode.PROMISE_IN_BOUNDS)


def _bsum(v):
    lanes = lax.iota(jnp.int32, LANES)
    for k in (8, 4, 2, 1):
        v = v + _perm(v, lanes ^ k)
    return v


def _ln(vs, gs, bs, n):
    s1 = vs[0]
    s2 = vs[0] * vs[0]
    for v in vs[1:]:
        s1 = s1 + v
        s2 = s2 + v * v
    s1 = _bsum(s1)[0]
    s2 = _bsum(s2)[0]
    m = s1 * (1.0 / n)
    var = s2 * (1.0 / n) - m * m
    rs = _rsqrt(var + EPS)
    c = -m * rs
    return [(v * rs + c) * g + b for v, g, b in zip(vs, gs, bs)]


def _sc_body(nblk, idxcat, numf, pt, pq, pg, p4, pars, out,
             idxv0, idxv1, rows0a, rows1a, rows2a, rows3a,
             rows0b, rows1b, rows2b, rows3b,
             numva, numvb, parv, outva, outvb,
             semi0, semi1, semo0, semo1):
    wid = lax.axis_index("c") * NS + lax.axis_index("s")
    tw = nblk * B0

    pltpu.sync_copy(pars, parv)
    ld = lambda o: parv[pl.ds(o, LANES)]
    w00, w01 = ld(0), ld(16)
    w10, w11 = ld(32), ld(48)
    w20, w21 = ld(64), ld(80)
    gc = (ld(96), ld(112))
    bc = (ld(128), ld(144))
    gn = (ld(160), ld(176))
    bn = (ld(192), ld(208))
    go = (ld(224), ld(240), ld(256), ld(272))
    bo = (ld(288), ld(304), ld(320), ld(336))

    slots = (
        (idxv0, (rows0a, rows1a, rows2a, rows3a), numva, outva, semi0, semo0),
        (idxv1, (rows0b, rows1b, rows2b, rows3b), numvb, outvb, semi1, semo1),
    )
    tables = (pt, pq, pg, p4)

    def in_copies(g, s):
        idxv, rows, numv, _, semi, _ = slots[s]
        base = wid * tw + g * B0
        cps = [pltpu.make_async_copy(tables[k].at[idxv.at[k]], rows[k], semi)
               for k in range(4)]
        cps.append(pltpu.make_async_copy(
            numf.at[pl.ds(3 * base, 3 * B0)], numv.at[pl.ds(0, 3 * B0)], semi))
        return cps

    def fire(g, s):
        idxv, _, _, _, _, _ = slots[s]
        base = wid * tw + g * B0
        pltpu.sync_copy(idxcat.at[:, pl.ds(base, B0)], idxv)
        for c in in_copies(g, s):
            c.start()

    def drain_in(g, s):
        for c in in_copies(g, s):
            c.wait()

    def out_copy(g, s):
        _, _, _, outv, _, semo = slots[s]
        base = wid * tw + g * B0
        return pltpu.make_async_copy(outv, out.at[pl.ds(base, B0), :], semo)

    def compute(g, s):
        _, rows, numv, outv, _, _ = slots[s]
        rows0, rows1, rows2, rows3 = rows

        def token(t):
            a0 = (rows0[t, pl.ds(0, LANES)] + rows1[t, pl.ds(0, LANES)]
                  + rows2[t, pl.ds(0, LANES)] + rows3[t, pl.ds(0, LANES)])
            a1 = (rows0[t, pl.ds(LANES, LANES)] + rows1[t, pl.ds(LANES, LANES)]
                  + rows2[t, pl.ds(LANES, LANES)] + rows3[t, pl.ds(LANES, LANES)])
            cat0, cat1 = _ln([a0, a1], gc, bc, HALF)
            nv = numv[pl.ds(3 * t, LANES)]
            n0, n1, n2 = nv[0], nv[1], nv[2]
            u0 = n0 * w00 + n1 * w10 + n2 * w20
            u1 = n0 * w01 + n1 * w11 + n2 * w21
            nm0, nm1 = _ln([u0, u1], gn, bn, HALF)
            o = _ln([cat0, cat1, nm0, nm1], go, bo, HID)
            outv[t, pl.ds(0, LANES)] = o[0]
            outv[t, pl.ds(LANES, LANES)] = o[1]
            outv[t, pl.ds(2 * LANES, LANES)] = o[2]
            outv[t, pl.ds(3 * LANES, LANES)] = o[3]

        plsc.parallel_loop(0, B0, 1, unroll=4)(token)
        out_copy(g, s).start()

    npair = nblk // 2
    fire(0, 0)

    def pair(i, _):
        g0 = 2 * i
        fire(g0 + 1, 1)
        drain_in(g0, 0)

        @pl.when(i > 0)
        def _():
            out_copy(g0, 0).wait()
        compute(g0, 0)

        @pl.when(i < npair - 1)
        def _():
            fire(g0 + 2, 0)
        drain_in(g0 + 1, 1)

        @pl.when(i > 0)
        def _():
            out_copy(g0 + 1, 1).wait()
        compute(g0 + 1, 1)
        return 0

    lax.fori_loop(0, npair, pair, 0)
    out_copy(nblk - 2, 0).wait()
    out_copy(nblk - 1, 1).wait()


def kernel(past_test, past_question, past_tag, past_testTag, num0, num1, num2,
           T_test, T_q, T_tag, T_tt, W_cat, b_cat, g_cat, beta_cat,
           W_num, b_num, g_num, beta_num, g_out, beta_out):
    B, L = past_test.shape
    T = B * L
    assert T % (NW * B0) == 0
    nblk = T // (NW * B0)

    pt, pq, pg, p4 = _project_tables(T_test, T_q, T_tag, T_tt, W_cat)

    idxcat = jnp.stack([
        past_test.reshape(T), past_question.reshape(T),
        past_tag.reshape(T), past_testTag.reshape(T),
    ])
    numf = jnp.concatenate([
        num0.astype(jnp.float32).reshape(T),
        num1.astype(jnp.float32).reshape(T),
        num2.astype(jnp.float32).reshape(T),
    ])
    pars = jnp.concatenate([
        W_num.astype(jnp.float32).reshape(3 * HALF),
        g_cat, beta_cat, g_num, beta_num, g_out, beta_out,
    ])

    mesh = plsc.VectorSubcoreMesh(core_axis_name="c", subcore_axis_name="s",
                                  num_cores=NC, num_subcores=NS)
    run = pl.kernel(
        functools.partial(_sc_body, nblk),
        out_type=jax.ShapeDtypeStruct((T, HID), jnp.float32),
        mesh=mesh,
        compiler_params=pltpu.CompilerParams(use_tc_tiling_on_sc=False),
        scratch_types=(
            [pltpu.VMEM((4, B0), jnp.int32)] * 2
            + [pltpu.VMEM((B0, HALF), jnp.float32)] * 8
            + [pltpu.VMEM((3 * B0 + LANES,), jnp.float32)] * 2
            + [pltpu.VMEM((11 * 32,), jnp.float32)]
            + [pltpu.VMEM((B0, HID), jnp.float32)] * 2
            + [pltpu.SemaphoreType.DMA] * 4
        ),
    )
    out = run(idxcat, numf, pt, pq, pg, p4, pars)
    return out.reshape(B, L, HID)
